# Initial kernel scaffold; baseline (speedup 1.0000x reference)
import jax
import jax.numpy as jnp
from jax.experimental import pallas as pl


def kernel(x, edge_index, W1, a_src1, a_dst1, b1, W2, a_src2, a_dst2, b2):
    raise NotImplementedError("write your pallas kernel here")


# speedup vs baseline: 22.7293x; 22.7293x over previous
import dataclasses
import functools

import jax
import jax.numpy as jnp
from jax import lax
from jax.experimental import pallas as pl
from jax.experimental.pallas import tpu as pltpu
from jax.experimental.pallas import tpu_sc as plsc

_L = 16
_K = 80
_NC = 2
_NS = 16
_NW = _NC * _NS


def _leaky(a):
    return jnp.maximum(a, 0.2 * a)


def _proj_body(x_ref, w_ref, av_ref, bv_ref, h_ref, s_ref, t_ref, mx_ref):
    h = jnp.dot(x_ref[...], w_ref[...], preferred_element_type=jnp.float32)
    h_ref[...] = h
    sv = jnp.dot(h, av_ref[...], preferred_element_type=jnp.float32)
    tv = jnp.dot(h, bv_ref[...], preferred_element_type=jnp.float32)
    s_ref[...] = sv
    t_ref[...] = tv
    m = jnp.concatenate(
        [jnp.max(sv).reshape(1, 1), jnp.max(tv).reshape(1, 1)], axis=1)

    @pl.when(pl.program_id(0) == 0)
    def _():
        mx_ref[...] = m

    @pl.when(pl.program_id(0) != 0)
    def _():
        mx_ref[...] = jnp.maximum(mx_ref[...], m)


def _proj(x, W, av, bv, R):
    N, Din = x.shape
    D = W.shape[1]
    return pl.pallas_call(
        _proj_body,
        grid=(N // R,),
        in_specs=[
            pl.BlockSpec((R, Din), lambda i: (i, 0)),
            pl.BlockSpec((Din, D), lambda i: (0, 0)),
            pl.BlockSpec((D, 1), lambda i: (0, 0)),
            pl.BlockSpec((D, 1), lambda i: (0, 0)),
        ],
        out_specs=[
            pl.BlockSpec((R, D), lambda i: (i, 0)),
            pl.BlockSpec((R, 1), lambda i: (i, 0)),
            pl.BlockSpec((R, 1), lambda i: (i, 0)),
            pl.BlockSpec((1, 2), lambda i: (0, 0)),
        ],
        out_shape=[
            jax.ShapeDtypeStruct((N, D), jnp.float32),
            jax.ShapeDtypeStruct((N, 1), jnp.float32),
            jax.ShapeDtypeStruct((N, 1), jnp.float32),
            jax.ShapeDtypeStruct((1, 2), jnp.float32),
        ],
    )(x, W, av, bv)


def _merge(n0_ref, n1_ref, d0_ref, d1_ref, s_ref, t_ref, c_ref, h_ref):
    a = s_ref[...] + t_ref[...]
    es = jnp.exp(_leaky(a) - c_ref[0, 0])
    num = n0_ref[...] + n1_ref[...] + es * h_ref[...]
    den = d0_ref[...] + d1_ref[...] + es
    return num / den


def _combine_proj_body(n0_ref, n1_ref, d0_ref, d1_ref, s_ref, t_ref, c_ref,
                       h_ref, b_ref, w_ref, av_ref, bv_ref, h2_ref, s2_ref,
                       t2_ref, mx_ref):
    out1 = _merge(n0_ref, n1_ref, d0_ref, d1_ref, s_ref, t_ref, c_ref, h_ref)
    x2 = jnp.maximum(out1 + b_ref[...], 0.0)
    h2 = jnp.dot(x2, w_ref[...], preferred_element_type=jnp.float32)
    h2_ref[...] = h2
    sv = jnp.dot(h2, av_ref[...], preferred_element_type=jnp.float32)
    tv = jnp.dot(h2, bv_ref[...], preferred_element_type=jnp.float32)
    s2_ref[...] = sv
    t2_ref[...] = tv
    m = jnp.concatenate(
        [jnp.max(sv).reshape(1, 1), jnp.max(tv).reshape(1, 1)], axis=1)

    @pl.when(pl.program_id(0) == 0)
    def _():
        mx_ref[...] = m

    @pl.when(pl.program_id(0) != 0)
    def _():
        mx_ref[...] = jnp.maximum(mx_ref[...], m)


def _combine_proj(n0, n1, d0, d1, s, t, c, h, b, W, av, bv, R):
    N, D = h.shape
    D2 = W.shape[1]
    return pl.pallas_call(
        _combine_proj_body,
        grid=(N // R,),
        in_specs=[
            pl.BlockSpec((R, D), lambda i: (i, 0)),
            pl.BlockSpec((R, D), lambda i: (i, 0)),
            pl.BlockSpec((R, 1), lambda i: (i, 0)),
            pl.BlockSpec((R, 1), lambda i: (i, 0)),
            pl.BlockSpec((R, 1), lambda i: (i, 0)),
            pl.BlockSpec((R, 1), lambda i: (i, 0)),
            pl.BlockSpec((1, 1), lambda i: (0, 0)),
            pl.BlockSpec((R, D), lambda i: (i, 0)),
            pl.BlockSpec((1, D), lambda i: (0, 0)),
            pl.BlockSpec((D, D2), lambda i: (0, 0)),
            pl.BlockSpec((D2, 1), lambda i: (0, 0)),
            pl.BlockSpec((D2, 1), lambda i: (0, 0)),
        ],
        out_specs=[
            pl.BlockSpec((R, D2), lambda i: (i, 0)),
            pl.BlockSpec((R, 1), lambda i: (i, 0)),
            pl.BlockSpec((R, 1), lambda i: (i, 0)),
            pl.BlockSpec((1, 2), lambda i: (0, 0)),
        ],
        out_shape=[
            jax.ShapeDtypeStruct((N, D2), jnp.float32),
            jax.ShapeDtypeStruct((N, 1), jnp.float32),
            jax.ShapeDtypeStruct((N, 1), jnp.float32),
            jax.ShapeDtypeStruct((1, 2), jnp.float32),
        ],
    )(n0, n1, d0, d1, s, t, c, h, b, W, av, bv)


def _final_body(n0_ref, n1_ref, d0_ref, d1_ref, s_ref, t_ref, c_ref, h_ref,
                b_ref, o_ref):
    o_ref[...] = _merge(n0_ref, n1_ref, d0_ref, d1_ref, s_ref, t_ref, c_ref,
                        h_ref) + b_ref[...]


def _final(n0, n1, d0, d1, s, t, c, h, b, R):
    N, D = h.shape
    return pl.pallas_call(
        _final_body,
        grid=(N // R,),
        in_specs=[
            pl.BlockSpec((R, D), lambda i: (i, 0)),
            pl.BlockSpec((R, D), lambda i: (i, 0)),
            pl.BlockSpec((R, 1), lambda i: (i, 0)),
            pl.BlockSpec((R, 1), lambda i: (i, 0)),
            pl.BlockSpec((R, 1), lambda i: (i, 0)),
            pl.BlockSpec((R, 1), lambda i: (i, 0)),
            pl.BlockSpec((1, 1), lambda i: (0, 0)),
            pl.BlockSpec((R, D), lambda i: (i, 0)),
            pl.BlockSpec((1, D), lambda i: (0, 0)),
        ],
        out_specs=pl.BlockSpec((R, D), lambda i: (i, 0)),
        out_shape=jax.ShapeDtypeStruct((N, D), jnp.float32),
    )(n0, n1, d0, d1, s, t, c, h, b)


def _edge_aggregate(src, dst, asrc, adst, cvec, h):
    N, D = h.shape
    E = src.shape[0]
    epw = E // _NW
    nchunks = epw // _K
    stripe = 640
    last = N - 15 * stripe
    dden = N // 10
    dfull = dden // _K
    drem = dden - dfull * _K
    mesh = plsc.VectorSubcoreMesh(core_axis_name="c", subcore_axis_name="s")
    cp = pltpu.CompilerParams()
    if "needs_layout_passes" in pltpu.CompilerParams.__dataclass_fields__:
        cp = dataclasses.replace(cp, needs_layout_passes=False)

    @functools.partial(
        pl.kernel,
        compiler_params=cp,
        out_type=[
            jax.ShapeDtypeStruct((N, D), jnp.float32),
            jax.ShapeDtypeStruct((N, D), jnp.float32),
            jax.ShapeDtypeStruct((N,), jnp.float32),
            jax.ShapeDtypeStruct((N,), jnp.float32),
        ],
        mesh=mesh,
        scratch_types=[
            pltpu.VMEM((N,), jnp.float32),
            pltpu.VMEM((N,), jnp.float32),
            pltpu.VMEM((_K,), jnp.int32),
            pltpu.VMEM((_K,), jnp.int32),
            pltpu.VMEM((_K, D), jnp.float32),
            pltpu.VMEM((_K,), jnp.float32),
            pltpu.VMEM((_L,), jnp.float32),
            pltpu.VMEM_SHARED((N, D), jnp.float32),
            pltpu.VMEM_SHARED((N,), jnp.float32),
        ],
    )
    def edge_kernel(src_hbm, dst_hbm, asrc_hbm, adst_hbm, cvec_hbm, h_hbm,
                    num0_hbm, num1_hbm, den0_hbm, den1_hbm, asrc_v, adst_v,
                    sidx_v, didx_v, rows_v, ee_v, cvec_v, num_sh, den_sh):
        ci = lax.axis_index("c")
        si = lax.axis_index("s")
        wid = ci * _NS + si

        @pl.loop(0, _K)
        def _(r):
            for j in range(D // _L):
                rows_v[r, pl.ds(j * _L, _L)] = jnp.zeros((_L,), jnp.float32)

        @pl.loop(0, _K, step=_L)
        def _(i):
            ee_v[pl.ds(i, _L)] = jnp.zeros((_L,), jnp.float32)

        r0 = si * stripe
        nzero = jnp.where(si < 15, stripe // _K, last // _K)

        @pl.loop(0, nzero)
        def _(q):
            pltpu.sync_copy(rows_v, num_sh.at[pl.ds(r0 + q * _K, _K)])

        @pl.when(si < 10)
        def _():
            d0 = si * dden

            @pl.loop(0, dfull)
            def _(q):
                pltpu.sync_copy(ee_v, den_sh.at[pl.ds(d0 + q * _K, _K)])

            pltpu.sync_copy(ee_v.at[pl.ds(0, drem)],
                            den_sh.at[pl.ds(d0 + dfull * _K, drem)])

        pltpu.sync_copy(asrc_hbm, asrc_v)
        pltpu.sync_copy(adst_hbm, adst_v)
        pltpu.sync_copy(cvec_hbm, cvec_v)
        plsc.subcore_barrier()
        cval = cvec_v[...]

        base = wid * epw

        @pl.loop(0, nchunks)
        def _(q):
            off = base + q * _K
            pltpu.sync_copy(src_hbm.at[pl.ds(off, _K)], sidx_v)
            pltpu.sync_copy(dst_hbm.at[pl.ds(off, _K)], didx_v)
            pltpu.sync_copy(h_hbm.at[sidx_v], rows_v)
            for g in range(_K // _L):
                sg = sidx_v[pl.ds(g * _L, _L)]
                dg = didx_v[pl.ds(g * _L, _L)]
                a = (plsc.load_gather(asrc_v, [sg]) +
                     plsc.load_gather(adst_v, [dg]))
                ee_v[pl.ds(g * _L, _L)] = jnp.exp(_leaky(a) - cval)

            @pl.loop(0, _K)
            def _(r):
                w = plsc.load_gather(ee_v, [jnp.full((_L,), r, jnp.int32)])
                for j in range(D // _L):
                    sl = pl.ds(j * _L, _L)
                    rows_v[r, sl] = rows_v[r, sl] * w

            pltpu.sync_copy(rows_v, num_sh.at[didx_v], add=True)
            pltpu.sync_copy(ee_v, den_sh.at[didx_v], add=True)

        plsc.subcore_barrier()

        for cc, (num_hbm, den_hbm) in enumerate(
                [(num0_hbm, den0_hbm), (num1_hbm, den1_hbm)]):
            @pl.when(jnp.logical_and(ci == cc, si < 15))
            def _():
                pltpu.sync_copy(num_sh.at[pl.ds(r0, stripe)],
                                num_hbm.at[pl.ds(r0, stripe)])

            @pl.when(jnp.logical_and(ci == cc, si == 15))
            def _():
                pltpu.sync_copy(num_sh.at[pl.ds(r0, last)],
                                num_hbm.at[pl.ds(r0, last)])

            @pl.when(jnp.logical_and(ci == cc, si == 15))
            def _():
                pltpu.sync_copy(den_sh, den_hbm)

    return edge_kernel(src, dst, asrc, adst, cvec, h)


def kernel(x, edge_index, W1, a_src1, a_dst1, b1, W2, a_src2, a_dst2, b2):
    N = x.shape[0]
    R = 2000
    src = edge_index[0]
    dst = edge_index[1]

    h1, s1, t1, mx1 = _proj(x, W1, a_src1.reshape(-1, 1),
                            a_dst1.reshape(-1, 1), R)
    c1 = _leaky(mx1[0, 0] + mx1[0, 1])
    n0, n1, d0, d1 = _edge_aggregate(src, dst, s1.reshape(-1), t1.reshape(-1),
                                     jnp.full((_L,), c1, jnp.float32), h1)
    h2, s2, t2, mx2 = _combine_proj(n0, n1, d0.reshape(N, 1),
                                    d1.reshape(N, 1), s1, t1,
                                    c1.reshape(1, 1), h1, b1.reshape(1, -1),
                                    W2, a_src2.reshape(-1, 1),
                                    a_dst2.reshape(-1, 1), R)
    c2 = _leaky(mx2[0, 0] + mx2[0, 1])
    n0, n1, d0, d1 = _edge_aggregate(src, dst, s2.reshape(-1), t2.reshape(-1),
                                     jnp.full((_L,), c2, jnp.float32), h2)
    return _final(n0, n1, d0.reshape(N, 1), d1.reshape(N, 1), s2, t2,
                  c2.reshape(1, 1), h2, b2.reshape(1, -1), R)


# speedup vs baseline: 29.5083x; 1.2983x over previous
import dataclasses
import functools

import jax
import jax.numpy as jnp
from jax import lax
from jax.experimental import pallas as pl
from jax.experimental.pallas import tpu as pltpu
from jax.experimental.pallas import tpu_sc as plsc

_L = 16
_K = 80
_NC = 2
_NS = 16
_NW = _NC * _NS


def _leaky(a):
    return jnp.maximum(a, 0.2 * a)


def _proj_body(x_ref, w_ref, av_ref, bv_ref, h_ref, s_ref, t_ref, mx_ref):
    h = jnp.dot(x_ref[...], w_ref[...], preferred_element_type=jnp.float32)
    h_ref[...] = h
    sv = jnp.dot(h, av_ref[...], preferred_element_type=jnp.float32)
    tv = jnp.dot(h, bv_ref[...], preferred_element_type=jnp.float32)
    s_ref[...] = sv
    t_ref[...] = tv
    m = jnp.concatenate(
        [jnp.max(sv).reshape(1, 1), jnp.max(tv).reshape(1, 1)], axis=1)

    @pl.when(pl.program_id(0) == 0)
    def _():
        mx_ref[...] = m

    @pl.when(pl.program_id(0) != 0)
    def _():
        mx_ref[...] = jnp.maximum(mx_ref[...], m)


def _proj(x, W, av, bv, R):
    N, Din = x.shape
    D = W.shape[1]
    return pl.pallas_call(
        _proj_body,
        grid=(N // R,),
        in_specs=[
            pl.BlockSpec((R, Din), lambda i: (i, 0)),
            pl.BlockSpec((Din, D), lambda i: (0, 0)),
            pl.BlockSpec((D, 1), lambda i: (0, 0)),
            pl.BlockSpec((D, 1), lambda i: (0, 0)),
        ],
        out_specs=[
            pl.BlockSpec((R, D), lambda i: (i, 0)),
            pl.BlockSpec((R, 1), lambda i: (i, 0)),
            pl.BlockSpec((R, 1), lambda i: (i, 0)),
            pl.BlockSpec((1, 2), lambda i: (0, 0)),
        ],
        out_shape=[
            jax.ShapeDtypeStruct((N, D), jnp.float32),
            jax.ShapeDtypeStruct((N, 1), jnp.float32),
            jax.ShapeDtypeStruct((N, 1), jnp.float32),
            jax.ShapeDtypeStruct((1, 2), jnp.float32),
        ],
    )(x, W, av, bv)


def _merge(n0_ref, n1_ref, d0_ref, d1_ref, s_ref, t_ref, c_ref, h_ref):
    a = s_ref[...] + t_ref[...]
    es = jnp.exp(_leaky(a) - c_ref[0, 0])
    num = n0_ref[...] + n1_ref[...] + es * h_ref[...]
    den = d0_ref[...] + d1_ref[...] + es
    return num / den


def _combine_proj_body(n0_ref, n1_ref, d0_ref, d1_ref, s_ref, t_ref, c_ref,
                       h_ref, b_ref, w_ref, av_ref, bv_ref, h2_ref, s2_ref,
                       t2_ref, mx_ref):
    out1 = _merge(n0_ref, n1_ref, d0_ref, d1_ref, s_ref, t_ref, c_ref, h_ref)
    x2 = jnp.maximum(out1 + b_ref[...], 0.0)
    h2 = jnp.dot(x2, w_ref[...], preferred_element_type=jnp.float32)
    h2_ref[...] = h2
    sv = jnp.dot(h2, av_ref[...], preferred_element_type=jnp.float32)
    tv = jnp.dot(h2, bv_ref[...], preferred_element_type=jnp.float32)
    s2_ref[...] = sv
    t2_ref[...] = tv
    m = jnp.concatenate(
        [jnp.max(sv).reshape(1, 1), jnp.max(tv).reshape(1, 1)], axis=1)

    @pl.when(pl.program_id(0) == 0)
    def _():
        mx_ref[...] = m

    @pl.when(pl.program_id(0) != 0)
    def _():
        mx_ref[...] = jnp.maximum(mx_ref[...], m)


def _combine_proj(n0, n1, d0, d1, s, t, c, h, b, W, av, bv, R):
    N, D = h.shape
    D2 = W.shape[1]
    return pl.pallas_call(
        _combine_proj_body,
        grid=(N // R,),
        in_specs=[
            pl.BlockSpec((R, D), lambda i: (i, 0)),
            pl.BlockSpec((R, D), lambda i: (i, 0)),
            pl.BlockSpec((R, 1), lambda i: (i, 0)),
            pl.BlockSpec((R, 1), lambda i: (i, 0)),
            pl.BlockSpec((R, 1), lambda i: (i, 0)),
            pl.BlockSpec((R, 1), lambda i: (i, 0)),
            pl.BlockSpec((1, 1), lambda i: (0, 0)),
            pl.BlockSpec((R, D), lambda i: (i, 0)),
            pl.BlockSpec((1, D), lambda i: (0, 0)),
            pl.BlockSpec((D, D2), lambda i: (0, 0)),
            pl.BlockSpec((D2, 1), lambda i: (0, 0)),
            pl.BlockSpec((D2, 1), lambda i: (0, 0)),
        ],
        out_specs=[
            pl.BlockSpec((R, D2), lambda i: (i, 0)),
            pl.BlockSpec((R, 1), lambda i: (i, 0)),
            pl.BlockSpec((R, 1), lambda i: (i, 0)),
            pl.BlockSpec((1, 2), lambda i: (0, 0)),
        ],
        out_shape=[
            jax.ShapeDtypeStruct((N, D2), jnp.float32),
            jax.ShapeDtypeStruct((N, 1), jnp.float32),
            jax.ShapeDtypeStruct((N, 1), jnp.float32),
            jax.ShapeDtypeStruct((1, 2), jnp.float32),
        ],
    )(n0, n1, d0, d1, s, t, c, h, b, W, av, bv)


def _final_body(n0_ref, n1_ref, d0_ref, d1_ref, s_ref, t_ref, c_ref, h_ref,
                b_ref, o_ref):
    o_ref[...] = _merge(n0_ref, n1_ref, d0_ref, d1_ref, s_ref, t_ref, c_ref,
                        h_ref) + b_ref[...]


def _final(n0, n1, d0, d1, s, t, c, h, b, R):
    N, D = h.shape
    return pl.pallas_call(
        _final_body,
        grid=(N // R,),
        in_specs=[
            pl.BlockSpec((R, D), lambda i: (i, 0)),
            pl.BlockSpec((R, D), lambda i: (i, 0)),
            pl.BlockSpec((R, 1), lambda i: (i, 0)),
            pl.BlockSpec((R, 1), lambda i: (i, 0)),
            pl.BlockSpec((R, 1), lambda i: (i, 0)),
            pl.BlockSpec((R, 1), lambda i: (i, 0)),
            pl.BlockSpec((1, 1), lambda i: (0, 0)),
            pl.BlockSpec((R, D), lambda i: (i, 0)),
            pl.BlockSpec((1, D), lambda i: (0, 0)),
        ],
        out_specs=pl.BlockSpec((R, D), lambda i: (i, 0)),
        out_shape=jax.ShapeDtypeStruct((N, D), jnp.float32),
    )(n0, n1, d0, d1, s, t, c, h, b)


def _edge_aggregate(src, dst, asrc, adst, cvec, h):
    N, D = h.shape
    E = src.shape[0]
    epw = E // _NW
    nchunks = epw // _K
    stripe = 640
    last = N - 15 * stripe
    dden = N // 10
    dfull = dden // _K
    drem = dden - dfull * _K
    mesh = plsc.VectorSubcoreMesh(core_axis_name="c", subcore_axis_name="s")
    cp = pltpu.CompilerParams()
    if "needs_layout_passes" in pltpu.CompilerParams.__dataclass_fields__:
        cp = dataclasses.replace(cp, needs_layout_passes=False)

    @functools.partial(
        pl.kernel,
        compiler_params=cp,
        out_type=[
            jax.ShapeDtypeStruct((N, D), jnp.float32),
            jax.ShapeDtypeStruct((N, D), jnp.float32),
            jax.ShapeDtypeStruct((N,), jnp.float32),
            jax.ShapeDtypeStruct((N,), jnp.float32),
        ],
        mesh=mesh,
        scratch_types=[
            pltpu.VMEM((N,), jnp.float32),
            pltpu.VMEM((N,), jnp.float32),
            pltpu.VMEM((_K,), jnp.int32),
            pltpu.VMEM((_K,), jnp.int32),
            pltpu.VMEM((_K,), jnp.int32),
            pltpu.VMEM((_K,), jnp.int32),
            pltpu.VMEM((_K, D), jnp.float32),
            pltpu.VMEM((_K, D), jnp.float32),
            pltpu.VMEM((_K,), jnp.float32),
            pltpu.VMEM((_K,), jnp.float32),
            pltpu.VMEM((_L,), jnp.float32),
            pltpu.VMEM_SHARED((N, D), jnp.float32),
            pltpu.VMEM_SHARED((N,), jnp.float32),
            pltpu.SemaphoreType.DMA,
            pltpu.SemaphoreType.DMA,
            pltpu.SemaphoreType.DMA,
            pltpu.SemaphoreType.DMA,
        ],
    )
    def edge_kernel(src_hbm, dst_hbm, asrc_hbm, adst_hbm, cvec_hbm, h_hbm,
                    num0_hbm, num1_hbm, den0_hbm, den1_hbm, asrc_v, adst_v,
                    sidx_a, didx_a, sidx_b, didx_b, rows_a, rows_b, ee_a,
                    ee_b, cvec_v, num_sh, den_sh, sem_ia, sem_ib, sem_ga,
                    sem_gb):
        ci = lax.axis_index("c")
        si = lax.axis_index("s")
        wid = ci * _NS + si

        @pl.loop(0, _K)
        def _(r):
            for j in range(D // _L):
                rows_a[r, pl.ds(j * _L, _L)] = jnp.zeros((_L,), jnp.float32)

        @pl.loop(0, _K, step=_L)
        def _(i):
            ee_a[pl.ds(i, _L)] = jnp.zeros((_L,), jnp.float32)

        r0 = si * stripe
        nzero = jnp.where(si < 15, stripe // _K, last // _K)

        @pl.loop(0, nzero)
        def _(q):
            pltpu.sync_copy(rows_a, num_sh.at[pl.ds(r0 + q * _K, _K)])

        @pl.when(si < 10)
        def _():
            d0 = si * dden

            @pl.loop(0, dfull)
            def _(q):
                pltpu.sync_copy(ee_a, den_sh.at[pl.ds(d0 + q * _K, _K)])

            pltpu.sync_copy(ee_a.at[pl.ds(0, drem)],
                            den_sh.at[pl.ds(d0 + dfull * _K, drem)])

        pltpu.sync_copy(asrc_hbm, asrc_v)
        pltpu.sync_copy(adst_hbm, adst_v)
        pltpu.sync_copy(cvec_hbm, cvec_v)
        plsc.subcore_barrier()
        cval = cvec_v[...]

        base = wid * epw

        def issue_idx(sx, dx, sem, cq):
            off = base + jnp.minimum(cq, nchunks - 1) * _K
            pltpu.async_copy(src_hbm.at[pl.ds(off, _K)], sx, sem)
            pltpu.async_copy(dst_hbm.at[pl.ds(off, _K)], dx, sem)

        def wait_idx(sx, dx, sem):
            pltpu.make_async_copy(src_hbm.at[pl.ds(0, _K)], sx, sem).wait()
            pltpu.make_async_copy(dst_hbm.at[pl.ds(0, _K)], dx, sem).wait()

        def process(sx, dx, rx, ex, six, sgx, sy, dy, ry, siy, sgy, cq,
                    tail):
            for g in range(_K // _L):
                sg = sx[pl.ds(g * _L, _L)]
                dg = dx[pl.ds(g * _L, _L)]
                a = (plsc.load_gather(asrc_v, [sg]) +
                     plsc.load_gather(adst_v, [dg]))
                ex[pl.ds(g * _L, _L)] = jnp.exp(_leaky(a) - cval)

            pltpu.make_async_copy(h_hbm.at[sx], rx, sgx).wait()

            @pl.loop(0, _K)
            def _(r):
                w = plsc.load_gather(ex, [jnp.full((_L,), r, jnp.int32)])
                for j in range(D // _L):
                    sl = pl.ds(j * _L, _L)
                    rx[r, sl] = rx[r, sl] * w

            pltpu.sync_copy(rx, num_sh.at[dx], add=True)
            pltpu.sync_copy(ex, den_sh.at[dx], add=True)
            if not tail:
                issue_idx(sx, dx, six, cq + 2)
                wait_idx(sy, dy, siy)
                pltpu.async_copy(h_hbm.at[sy], ry, sgy)

        issue_idx(sidx_a, didx_a, sem_ia, 0)
        issue_idx(sidx_b, didx_b, sem_ib, 1)
        wait_idx(sidx_a, didx_a, sem_ia)
        pltpu.async_copy(h_hbm.at[sidx_a], rows_a, sem_ga)

        @pl.loop(0, (nchunks - 1) // 2)
        def _(t):
            cq = t * 2
            process(sidx_a, didx_a, rows_a, ee_a, sem_ia, sem_ga,
                    sidx_b, didx_b, rows_b, sem_ib, sem_gb, cq, False)
            process(sidx_b, didx_b, rows_b, ee_b, sem_ib, sem_gb,
                    sidx_a, didx_a, rows_a, sem_ia, sem_ga, cq + 1, False)

        process(sidx_a, didx_a, rows_a, ee_a, sem_ia, sem_ga,
                sidx_b, didx_b, rows_b, sem_ib, sem_gb, nchunks - 1, True)
        wait_idx(sidx_b, didx_b, sem_ib)

        plsc.subcore_barrier()

        for cc, (num_hbm, den_hbm) in enumerate(
                [(num0_hbm, den0_hbm), (num1_hbm, den1_hbm)]):
            @pl.when(jnp.logical_and(ci == cc, si < 15))
            def _():
                pltpu.sync_copy(num_sh.at[pl.ds(r0, stripe)],
                                num_hbm.at[pl.ds(r0, stripe)])

            @pl.when(jnp.logical_and(ci == cc, si == 15))
            def _():
                pltpu.sync_copy(num_sh.at[pl.ds(r0, last)],
                                num_hbm.at[pl.ds(r0, last)])

            @pl.when(jnp.logical_and(ci == cc, si == 15))
            def _():
                pltpu.sync_copy(den_sh, den_hbm)

    return edge_kernel(src, dst, asrc, adst, cvec, h)


def kernel(x, edge_index, W1, a_src1, a_dst1, b1, W2, a_src2, a_dst2, b2):
    N = x.shape[0]
    R = 2000
    src = edge_index[0]
    dst = edge_index[1]

    h1, s1, t1, mx1 = _proj(x, W1, a_src1.reshape(-1, 1),
                            a_dst1.reshape(-1, 1), R)
    c1 = _leaky(mx1[0, 0] + mx1[0, 1])
    n0, n1, d0, d1 = _edge_aggregate(src, dst, s1.reshape(-1), t1.reshape(-1),
                                     jnp.full((_L,), c1, jnp.float32), h1)
    h2, s2, t2, mx2 = _combine_proj(n0, n1, d0.reshape(N, 1),
                                    d1.reshape(N, 1), s1, t1,
                                    c1.reshape(1, 1), h1, b1.reshape(1, -1),
                                    W2, a_src2.reshape(-1, 1),
                                    a_dst2.reshape(-1, 1), R)
    c2 = _leaky(mx2[0, 0] + mx2[0, 1])
    n0, n1, d0, d1 = _edge_aggregate(src, dst, s2.reshape(-1), t2.reshape(-1),
                                     jnp.full((_L,), c2, jnp.float32), h2)
    return _final(n0, n1, d0.reshape(N, 1), d1.reshape(N, 1), s2, t2,
                  c2.reshape(1, 1), h2, b2.reshape(1, -1), R)


# speedup vs baseline: 30.1594x; 1.0221x over previous
import dataclasses
import functools

import jax
import jax.numpy as jnp
from jax import lax
from jax.experimental import pallas as pl
from jax.experimental.pallas import tpu as pltpu
from jax.experimental.pallas import tpu_sc as plsc

_L = 16
_K = 80
_NC = 2
_NS = 16
_NW = _NC * _NS


def _leaky(a):
    return jnp.maximum(a, 0.2 * a)


def _proj_body(x_ref, w_ref, av_ref, bv_ref, h_ref, s_ref, t_ref, mx_ref):
    h = jnp.dot(x_ref[...], w_ref[...], preferred_element_type=jnp.float32)
    h_ref[...] = h
    sv = jnp.dot(h, av_ref[...], preferred_element_type=jnp.float32)
    tv = jnp.dot(h, bv_ref[...], preferred_element_type=jnp.float32)
    s_ref[...] = sv
    t_ref[...] = tv
    m = jnp.concatenate(
        [jnp.max(sv).reshape(1, 1), jnp.max(tv).reshape(1, 1)], axis=1)

    @pl.when(pl.program_id(0) == 0)
    def _():
        mx_ref[...] = m

    @pl.when(pl.program_id(0) != 0)
    def _():
        mx_ref[...] = jnp.maximum(mx_ref[...], m)


def _proj(x, W, av, bv, R):
    N, Din = x.shape
    D = W.shape[1]
    return pl.pallas_call(
        _proj_body,
        grid=(N // R,),
        in_specs=[
            pl.BlockSpec((R, Din), lambda i: (i, 0)),
            pl.BlockSpec((Din, D), lambda i: (0, 0)),
            pl.BlockSpec((D, 1), lambda i: (0, 0)),
            pl.BlockSpec((D, 1), lambda i: (0, 0)),
        ],
        out_specs=[
            pl.BlockSpec((R, D), lambda i: (i, 0)),
            pl.BlockSpec((R, 1), lambda i: (i, 0)),
            pl.BlockSpec((R, 1), lambda i: (i, 0)),
            pl.BlockSpec((1, 2), lambda i: (0, 0)),
        ],
        out_shape=[
            jax.ShapeDtypeStruct((N, D), jnp.float32),
            jax.ShapeDtypeStruct((N, 1), jnp.float32),
            jax.ShapeDtypeStruct((N, 1), jnp.float32),
            jax.ShapeDtypeStruct((1, 2), jnp.float32),
        ],
    )(x, W, av, bv)


def _merge(n0_ref, n1_ref, d0_ref, d1_ref, s_ref, t_ref, c_ref, h_ref):
    a = s_ref[...] + t_ref[...]
    es = jnp.exp(_leaky(a) - c_ref[0, 0])
    num = n0_ref[...] + n1_ref[...] + es * h_ref[...]
    den = d0_ref[...] + d1_ref[...] + es
    return num / den


def _combine_proj_body(n0_ref, n1_ref, d0_ref, d1_ref, s_ref, t_ref, c_ref,
                       h_ref, b_ref, w_ref, av_ref, bv_ref, h2_ref, s2_ref,
                       t2_ref, mx_ref):
    out1 = _merge(n0_ref, n1_ref, d0_ref, d1_ref, s_ref, t_ref, c_ref, h_ref)
    x2 = jnp.maximum(out1 + b_ref[...], 0.0)
    h2 = jnp.dot(x2, w_ref[...], preferred_element_type=jnp.float32)
    h2_ref[...] = h2
    sv = jnp.dot(h2, av_ref[...], preferred_element_type=jnp.float32)
    tv = jnp.dot(h2, bv_ref[...], preferred_element_type=jnp.float32)
    s2_ref[...] = sv
    t2_ref[...] = tv
    m = jnp.concatenate(
        [jnp.max(sv).reshape(1, 1), jnp.max(tv).reshape(1, 1)], axis=1)

    @pl.when(pl.program_id(0) == 0)
    def _():
        mx_ref[...] = m

    @pl.when(pl.program_id(0) != 0)
    def _():
        mx_ref[...] = jnp.maximum(mx_ref[...], m)


def _combine_proj(n0, n1, d0, d1, s, t, c, h, b, W, av, bv, R):
    N, D = h.shape
    D2 = W.shape[1]
    return pl.pallas_call(
        _combine_proj_body,
        grid=(N // R,),
        in_specs=[
            pl.BlockSpec((R, D), lambda i: (i, 0)),
            pl.BlockSpec((R, D), lambda i: (i, 0)),
            pl.BlockSpec((R, 1), lambda i: (i, 0)),
            pl.BlockSpec((R, 1), lambda i: (i, 0)),
            pl.BlockSpec((R, 1), lambda i: (i, 0)),
            pl.BlockSpec((R, 1), lambda i: (i, 0)),
            pl.BlockSpec((1, 1), lambda i: (0, 0)),
            pl.BlockSpec((R, D), lambda i: (i, 0)),
            pl.BlockSpec((1, D), lambda i: (0, 0)),
            pl.BlockSpec((D, D2), lambda i: (0, 0)),
            pl.BlockSpec((D2, 1), lambda i: (0, 0)),
            pl.BlockSpec((D2, 1), lambda i: (0, 0)),
        ],
        out_specs=[
            pl.BlockSpec((R, D2), lambda i: (i, 0)),
            pl.BlockSpec((R, 1), lambda i: (i, 0)),
            pl.BlockSpec((R, 1), lambda i: (i, 0)),
            pl.BlockSpec((1, 2), lambda i: (0, 0)),
        ],
        out_shape=[
            jax.ShapeDtypeStruct((N, D2), jnp.float32),
            jax.ShapeDtypeStruct((N, 1), jnp.float32),
            jax.ShapeDtypeStruct((N, 1), jnp.float32),
            jax.ShapeDtypeStruct((1, 2), jnp.float32),
        ],
    )(n0, n1, d0, d1, s, t, c, h, b, W, av, bv)


def _final_body(n0_ref, n1_ref, d0_ref, d1_ref, s_ref, t_ref, c_ref, h_ref,
                b_ref, o_ref):
    o_ref[...] = _merge(n0_ref, n1_ref, d0_ref, d1_ref, s_ref, t_ref, c_ref,
                        h_ref) + b_ref[...]


def _final(n0, n1, d0, d1, s, t, c, h, b, R):
    N, D = h.shape
    return pl.pallas_call(
        _final_body,
        grid=(N // R,),
        in_specs=[
            pl.BlockSpec((R, D), lambda i: (i, 0)),
            pl.BlockSpec((R, D), lambda i: (i, 0)),
            pl.BlockSpec((R, 1), lambda i: (i, 0)),
            pl.BlockSpec((R, 1), lambda i: (i, 0)),
            pl.BlockSpec((R, 1), lambda i: (i, 0)),
            pl.BlockSpec((R, 1), lambda i: (i, 0)),
            pl.BlockSpec((1, 1), lambda i: (0, 0)),
            pl.BlockSpec((R, D), lambda i: (i, 0)),
            pl.BlockSpec((1, D), lambda i: (0, 0)),
        ],
        out_specs=pl.BlockSpec((R, D), lambda i: (i, 0)),
        out_shape=jax.ShapeDtypeStruct((N, D), jnp.float32),
    )(n0, n1, d0, d1, s, t, c, h, b)


def _edge_aggregate(src, dst, asrc, adst, cvec, h):
    N, D = h.shape
    E = src.shape[0]
    epw = E // _NW
    nchunks = epw // _K
    stripe = 640
    last = N - 15 * stripe
    dden = N // 10
    dfull = dden // _K
    drem = dden - dfull * _K
    mesh = plsc.VectorSubcoreMesh(core_axis_name="c", subcore_axis_name="s")
    cp = pltpu.CompilerParams()
    if "needs_layout_passes" in pltpu.CompilerParams.__dataclass_fields__:
        cp = dataclasses.replace(cp, needs_layout_passes=False)

    @functools.partial(
        pl.kernel,
        compiler_params=cp,
        out_type=[
            jax.ShapeDtypeStruct((N, D), jnp.float32),
            jax.ShapeDtypeStruct((N, D), jnp.float32),
            jax.ShapeDtypeStruct((N,), jnp.float32),
            jax.ShapeDtypeStruct((N,), jnp.float32),
        ],
        mesh=mesh,
        scratch_types=[
            pltpu.VMEM((N,), jnp.float32),
            pltpu.VMEM((N,), jnp.float32),
            pltpu.VMEM((_K,), jnp.int32),
            pltpu.VMEM((_K,), jnp.int32),
            pltpu.VMEM((_K,), jnp.int32),
            pltpu.VMEM((_K,), jnp.int32),
            pltpu.VMEM((_K, D), jnp.float32),
            pltpu.VMEM((_K, D), jnp.float32),
            pltpu.VMEM((_K,), jnp.float32),
            pltpu.VMEM((_K,), jnp.float32),
            pltpu.VMEM((_L,), jnp.float32),
            pltpu.VMEM_SHARED((N, D), jnp.float32),
            pltpu.VMEM_SHARED((N,), jnp.float32),
            pltpu.SemaphoreType.DMA,
            pltpu.SemaphoreType.DMA,
            pltpu.SemaphoreType.DMA,
            pltpu.SemaphoreType.DMA,
        ],
    )
    def edge_kernel(src_hbm, dst_hbm, asrc_hbm, adst_hbm, cvec_hbm, h_hbm,
                    num0_hbm, num1_hbm, den0_hbm, den1_hbm, asrc_v, adst_v,
                    sidx_a, didx_a, sidx_b, didx_b, rows_a, rows_b, ee_a,
                    ee_b, cvec_v, num_sh, den_sh, sem_ia, sem_ib, sem_ga,
                    sem_gb):
        ci = lax.axis_index("c")
        si = lax.axis_index("s")
        wid = ci * _NS + si

        @pl.loop(0, _K)
        def _(r):
            for j in range(D // _L):
                rows_a[r, pl.ds(j * _L, _L)] = jnp.zeros((_L,), jnp.float32)

        @pl.loop(0, _K, step=_L)
        def _(i):
            ee_a[pl.ds(i, _L)] = jnp.zeros((_L,), jnp.float32)

        r0 = si * stripe
        nzero = jnp.where(si < 15, stripe // _K, last // _K)

        @pl.loop(0, nzero)
        def _(q):
            pltpu.sync_copy(rows_a, num_sh.at[pl.ds(r0 + q * _K, _K)])

        @pl.when(si < 10)
        def _():
            d0 = si * dden

            @pl.loop(0, dfull)
            def _(q):
                pltpu.sync_copy(ee_a, den_sh.at[pl.ds(d0 + q * _K, _K)])

            pltpu.sync_copy(ee_a.at[pl.ds(0, drem)],
                            den_sh.at[pl.ds(d0 + dfull * _K, drem)])

        pltpu.sync_copy(asrc_hbm, asrc_v)
        pltpu.sync_copy(adst_hbm, adst_v)
        pltpu.sync_copy(cvec_hbm, cvec_v)
        plsc.subcore_barrier()
        cval = cvec_v[...]

        base = wid * epw

        def issue_idx(sx, dx, sem, cq):
            off = base + jnp.minimum(cq, nchunks - 1) * _K
            pltpu.async_copy(src_hbm.at[pl.ds(off, _K)], sx, sem)
            pltpu.async_copy(dst_hbm.at[pl.ds(off, _K)], dx, sem)

        def wait_idx(sx, dx, sem):
            pltpu.make_async_copy(src_hbm.at[pl.ds(0, _K)], sx, sem).wait()
            pltpu.make_async_copy(dst_hbm.at[pl.ds(0, _K)], dx, sem).wait()

        def process(sx, dx, rx, ex, six, sgx, sy, dy, ry, siy, sgy, cq,
                    tail):
            for g in range(_K // _L):
                sg = sx[pl.ds(g * _L, _L)]
                dg = dx[pl.ds(g * _L, _L)]
                a = (plsc.load_gather(asrc_v, [sg]) +
                     plsc.load_gather(adst_v, [dg]))
                ex[pl.ds(g * _L, _L)] = jnp.exp(_leaky(a) - cval)

            pltpu.make_async_copy(h_hbm.at[sx], rx, sgx).wait()

            @pl.loop(0, _K // _L)
            def _(gi):
                r0v = gi * _L
                for i in range(_L):
                    w = plsc.load_gather(
                        ex, [jnp.full((_L,), r0v + i, jnp.int32)])
                    for j in range(D // _L):
                        sl = pl.ds(j * _L, _L)
                        rx[r0v + i, sl] = rx[r0v + i, sl] * w

            pltpu.sync_copy(rx, num_sh.at[dx], add=True)
            pltpu.sync_copy(ex, den_sh.at[dx], add=True)
            if not tail:
                issue_idx(sx, dx, six, cq + 2)
                wait_idx(sy, dy, siy)
                pltpu.async_copy(h_hbm.at[sy], ry, sgy)

        issue_idx(sidx_a, didx_a, sem_ia, 0)
        issue_idx(sidx_b, didx_b, sem_ib, 1)
        wait_idx(sidx_a, didx_a, sem_ia)
        pltpu.async_copy(h_hbm.at[sidx_a], rows_a, sem_ga)

        @pl.loop(0, (nchunks - 1) // 2)
        def _(t):
            cq = t * 2
            process(sidx_a, didx_a, rows_a, ee_a, sem_ia, sem_ga,
                    sidx_b, didx_b, rows_b, sem_ib, sem_gb, cq, False)
            process(sidx_b, didx_b, rows_b, ee_b, sem_ib, sem_gb,
                    sidx_a, didx_a, rows_a, sem_ia, sem_ga, cq + 1, False)

        process(sidx_a, didx_a, rows_a, ee_a, sem_ia, sem_ga,
                sidx_b, didx_b, rows_b, sem_ib, sem_gb, nchunks - 1, True)
        wait_idx(sidx_b, didx_b, sem_ib)

        plsc.subcore_barrier()

        for cc, (num_hbm, den_hbm) in enumerate(
                [(num0_hbm, den0_hbm), (num1_hbm, den1_hbm)]):
            @pl.when(jnp.logical_and(ci == cc, si < 15))
            def _():
                pltpu.sync_copy(num_sh.at[pl.ds(r0, stripe)],
                                num_hbm.at[pl.ds(r0, stripe)])

            @pl.when(jnp.logical_and(ci == cc, si == 15))
            def _():
                pltpu.sync_copy(num_sh.at[pl.ds(r0, last)],
                                num_hbm.at[pl.ds(r0, last)])

            @pl.when(jnp.logical_and(ci == cc, si == 15))
            def _():
                pltpu.sync_copy(den_sh, den_hbm)

    return edge_kernel(src, dst, asrc, adst, cvec, h)


def kernel(x, edge_index, W1, a_src1, a_dst1, b1, W2, a_src2, a_dst2, b2):
    N = x.shape[0]
    R = 2000
    src = edge_index[0]
    dst = edge_index[1]

    h1, s1, t1, mx1 = _proj(x, W1, a_src1.reshape(-1, 1),
                            a_dst1.reshape(-1, 1), R)
    c1 = _leaky(mx1[0, 0] + mx1[0, 1])
    n0, n1, d0, d1 = _edge_aggregate(src, dst, s1.reshape(-1), t1.reshape(-1),
                                     jnp.full((_L,), c1, jnp.float32), h1)
    h2, s2, t2, mx2 = _combine_proj(n0, n1, d0.reshape(N, 1),
                                    d1.reshape(N, 1), s1, t1,
                                    c1.reshape(1, 1), h1, b1.reshape(1, -1),
                                    W2, a_src2.reshape(-1, 1),
                                    a_dst2.reshape(-1, 1), R)
    c2 = _leaky(mx2[0, 0] + mx2[0, 1])
    n0, n1, d0, d1 = _edge_aggregate(src, dst, s2.reshape(-1), t2.reshape(-1),
                                     jnp.full((_L,), c2, jnp.float32), h2)
    return _final(n0, n1, d0.reshape(N, 1), d1.reshape(N, 1), s2, t2,
                  c2.reshape(1, 1), h2, b2.reshape(1, -1), R)


# speedup vs baseline: 36.6783x; 1.2161x over previous
import dataclasses
import functools

import jax
import jax.numpy as jnp
from jax import lax
from jax.experimental import pallas as pl
from jax.experimental.pallas import tpu as pltpu
from jax.experimental.pallas import tpu_sc as plsc

_L = 16
_K = 80
_NC = 2
_NS = 16
_NW = _NC * _NS


def _leaky(a):
    return jnp.maximum(a, 0.2 * a)


def _proj_body(x_ref, w_ref, av_ref, bv_ref, h_ref, s_ref, t_ref, mx_ref):
    h = jnp.dot(x_ref[...], w_ref[...], preferred_element_type=jnp.float32)
    h_ref[...] = h
    sv = jnp.dot(h, av_ref[...], preferred_element_type=jnp.float32)
    tv = jnp.dot(h, bv_ref[...], preferred_element_type=jnp.float32)
    s_ref[...] = sv
    t_ref[...] = tv
    m = jnp.concatenate(
        [jnp.max(sv).reshape(1, 1), jnp.max(tv).reshape(1, 1)], axis=1)

    @pl.when(pl.program_id(0) == 0)
    def _():
        mx_ref[...] = m

    @pl.when(pl.program_id(0) != 0)
    def _():
        mx_ref[...] = jnp.maximum(mx_ref[...], m)


def _proj(x, W, av, bv, R):
    N, Din = x.shape
    D = W.shape[1]
    return pl.pallas_call(
        _proj_body,
        grid=(N // R,),
        in_specs=[
            pl.BlockSpec((R, Din), lambda i: (i, 0)),
            pl.BlockSpec((Din, D), lambda i: (0, 0)),
            pl.BlockSpec((D, 1), lambda i: (0, 0)),
            pl.BlockSpec((D, 1), lambda i: (0, 0)),
        ],
        out_specs=[
            pl.BlockSpec((R, D), lambda i: (i, 0)),
            pl.BlockSpec((R, 1), lambda i: (i, 0)),
            pl.BlockSpec((R, 1), lambda i: (i, 0)),
            pl.BlockSpec((1, 2), lambda i: (0, 0)),
        ],
        out_shape=[
            jax.ShapeDtypeStruct((N, D), jnp.float32),
            jax.ShapeDtypeStruct((N, 1), jnp.float32),
            jax.ShapeDtypeStruct((N, 1), jnp.float32),
            jax.ShapeDtypeStruct((1, 2), jnp.float32),
        ],
    )(x, W, av, bv)


def _merge(n0_ref, n1_ref, d0_ref, d1_ref, s_ref, t_ref, c_ref, h_ref):
    a = s_ref[...] + t_ref[...]
    es = jnp.exp(_leaky(a) - c_ref[0, 0])
    num = n0_ref[...] + n1_ref[...] + es * h_ref[...]
    den = d0_ref[...] + d1_ref[...] + es
    return num / den


def _combine_proj_body(n0_ref, n1_ref, d0_ref, d1_ref, s_ref, t_ref, c_ref,
                       h_ref, b_ref, w_ref, av_ref, bv_ref, h2_ref, s2_ref,
                       t2_ref, mx_ref):
    out1 = _merge(n0_ref, n1_ref, d0_ref, d1_ref, s_ref, t_ref, c_ref, h_ref)
    x2 = jnp.maximum(out1 + b_ref[...], 0.0)
    h2 = jnp.dot(x2, w_ref[...], preferred_element_type=jnp.float32)
    h2_ref[...] = h2
    sv = jnp.dot(h2, av_ref[...], preferred_element_type=jnp.float32)
    tv = jnp.dot(h2, bv_ref[...], preferred_element_type=jnp.float32)
    s2_ref[...] = sv
    t2_ref[...] = tv
    m = jnp.concatenate(
        [jnp.max(sv).reshape(1, 1), jnp.max(tv).reshape(1, 1)], axis=1)

    @pl.when(pl.program_id(0) == 0)
    def _():
        mx_ref[...] = m

    @pl.when(pl.program_id(0) != 0)
    def _():
        mx_ref[...] = jnp.maximum(mx_ref[...], m)


def _combine_proj(n0, n1, d0, d1, s, t, c, h, b, W, av, bv, R):
    N, D = h.shape
    D2 = W.shape[1]
    return pl.pallas_call(
        _combine_proj_body,
        grid=(N // R,),
        in_specs=[
            pl.BlockSpec((R, D), lambda i: (i, 0)),
            pl.BlockSpec((R, D), lambda i: (i, 0)),
            pl.BlockSpec((R, 1), lambda i: (i, 0)),
            pl.BlockSpec((R, 1), lambda i: (i, 0)),
            pl.BlockSpec((R, 1), lambda i: (i, 0)),
            pl.BlockSpec((R, 1), lambda i: (i, 0)),
            pl.BlockSpec((1, 1), lambda i: (0, 0)),
            pl.BlockSpec((R, D), lambda i: (i, 0)),
            pl.BlockSpec((1, D), lambda i: (0, 0)),
            pl.BlockSpec((D, D2), lambda i: (0, 0)),
            pl.BlockSpec((D2, 1), lambda i: (0, 0)),
            pl.BlockSpec((D2, 1), lambda i: (0, 0)),
        ],
        out_specs=[
            pl.BlockSpec((R, D2), lambda i: (i, 0)),
            pl.BlockSpec((R, 1), lambda i: (i, 0)),
            pl.BlockSpec((R, 1), lambda i: (i, 0)),
            pl.BlockSpec((1, 2), lambda i: (0, 0)),
        ],
        out_shape=[
            jax.ShapeDtypeStruct((N, D2), jnp.float32),
            jax.ShapeDtypeStruct((N, 1), jnp.float32),
            jax.ShapeDtypeStruct((N, 1), jnp.float32),
            jax.ShapeDtypeStruct((1, 2), jnp.float32),
        ],
    )(n0, n1, d0, d1, s, t, c, h, b, W, av, bv)


def _final_body(n0_ref, n1_ref, d0_ref, d1_ref, s_ref, t_ref, c_ref, h_ref,
                b_ref, o_ref):
    o_ref[...] = _merge(n0_ref, n1_ref, d0_ref, d1_ref, s_ref, t_ref, c_ref,
                        h_ref) + b_ref[...]


def _final(n0, n1, d0, d1, s, t, c, h, b, R):
    N, D = h.shape
    return pl.pallas_call(
        _final_body,
        grid=(N // R,),
        in_specs=[
            pl.BlockSpec((R, D), lambda i: (i, 0)),
            pl.BlockSpec((R, D), lambda i: (i, 0)),
            pl.BlockSpec((R, 1), lambda i: (i, 0)),
            pl.BlockSpec((R, 1), lambda i: (i, 0)),
            pl.BlockSpec((R, 1), lambda i: (i, 0)),
            pl.BlockSpec((R, 1), lambda i: (i, 0)),
            pl.BlockSpec((1, 1), lambda i: (0, 0)),
            pl.BlockSpec((R, D), lambda i: (i, 0)),
            pl.BlockSpec((1, D), lambda i: (0, 0)),
        ],
        out_specs=pl.BlockSpec((R, D), lambda i: (i, 0)),
        out_shape=jax.ShapeDtypeStruct((N, D), jnp.float32),
    )(n0, n1, d0, d1, s, t, c, h, b)


def _edge_aggregate(src, dst, asrc, adst, cvec, h):
    N, D = h.shape
    E = src.shape[0]
    epw = E // _NW
    nchunks = epw // _K
    stripe = 640
    last = N - 15 * stripe
    dden = N // 10
    dfull = dden // _K
    drem = dden - dfull * _K
    mesh = plsc.VectorSubcoreMesh(core_axis_name="c", subcore_axis_name="s")
    cp = pltpu.CompilerParams()
    if "needs_layout_passes" in pltpu.CompilerParams.__dataclass_fields__:
        cp = dataclasses.replace(cp, needs_layout_passes=False)

    @functools.partial(
        pl.kernel,
        compiler_params=cp,
        out_type=[
            jax.ShapeDtypeStruct((N, D), jnp.float32),
            jax.ShapeDtypeStruct((N, D), jnp.float32),
            jax.ShapeDtypeStruct((N,), jnp.float32),
            jax.ShapeDtypeStruct((N,), jnp.float32),
        ],
        mesh=mesh,
        scratch_types=[
            pltpu.VMEM((N,), jnp.float32),
            pltpu.VMEM((N,), jnp.float32),
            pltpu.VMEM((_K,), jnp.int32),
            pltpu.VMEM((_K,), jnp.int32),
            pltpu.VMEM((_K,), jnp.int32),
            pltpu.VMEM((_K,), jnp.int32),
            pltpu.VMEM((_K, D), jnp.float32),
            pltpu.VMEM((_K, D), jnp.float32),
            pltpu.VMEM((_K,), jnp.float32),
            pltpu.VMEM((_K,), jnp.float32),
            pltpu.VMEM((_L,), jnp.float32),
            pltpu.VMEM_SHARED((N, D), jnp.float32),
            pltpu.VMEM_SHARED((N,), jnp.float32),
            pltpu.SemaphoreType.DMA,
            pltpu.SemaphoreType.DMA,
            pltpu.SemaphoreType.DMA,
            pltpu.SemaphoreType.DMA,
        ],
    )
    def edge_kernel(src_hbm, dst_hbm, asrc_hbm, adst_hbm, cvec_hbm, h_hbm,
                    num0_hbm, num1_hbm, den0_hbm, den1_hbm, asrc_v, adst_v,
                    sidx_a, didx_a, sidx_b, didx_b, rows_a, rows_b, ee_a,
                    ee_b, cvec_v, num_sh, den_sh, sem_ia, sem_ib, sem_ga,
                    sem_gb):
        ci = lax.axis_index("c")
        si = lax.axis_index("s")
        wid = ci * _NS + si

        @pl.loop(0, _K)
        def _(r):
            for j in range(D // _L):
                rows_a[r, pl.ds(j * _L, _L)] = jnp.zeros((_L,), jnp.float32)

        @pl.loop(0, _K, step=_L)
        def _(i):
            ee_a[pl.ds(i, _L)] = jnp.zeros((_L,), jnp.float32)

        r0 = si * stripe
        nzero = jnp.where(si < 15, stripe // _K, last // _K)

        @pl.loop(0, nzero)
        def _(q):
            pltpu.sync_copy(rows_a, num_sh.at[pl.ds(r0 + q * _K, _K)])

        @pl.when(si < 10)
        def _():
            d0 = si * dden

            @pl.loop(0, dfull)
            def _(q):
                pltpu.sync_copy(ee_a, den_sh.at[pl.ds(d0 + q * _K, _K)])

            pltpu.sync_copy(ee_a.at[pl.ds(0, drem)],
                            den_sh.at[pl.ds(d0 + dfull * _K, drem)])

        pltpu.sync_copy(asrc_hbm, asrc_v)
        pltpu.sync_copy(adst_hbm, adst_v)
        pltpu.sync_copy(cvec_hbm, cvec_v)
        plsc.subcore_barrier()
        cval = cvec_v[...]

        base = wid * epw

        def issue_idx(sx, dx, sem, cq):
            off = base + jnp.minimum(cq, nchunks - 1) * _K
            pltpu.async_copy(src_hbm.at[pl.ds(off, _K)], sx, sem)
            pltpu.async_copy(dst_hbm.at[pl.ds(off, _K)], dx, sem)

        def wait_idx(sx, dx, sem):
            pltpu.make_async_copy(src_hbm.at[pl.ds(0, _K)], sx, sem).wait()
            pltpu.make_async_copy(dst_hbm.at[pl.ds(0, _K)], dx, sem).wait()

        def process(sx, dx, rx, ex, six, sgx, sy, dy, ry, siy, sgy, cq,
                    tail):
            pltpu.make_async_copy(h_hbm.at[sx], rx, sgx).wait()

            if not tail:
                wait_idx(sy, dy, siy)
                pltpu.async_copy(h_hbm.at[sy], ry, sgy)

            for g in range(_K // _L):
                sg = sx[pl.ds(g * _L, _L)]
                dg = dx[pl.ds(g * _L, _L)]
                a = (plsc.load_gather(asrc_v, [sg]) +
                     plsc.load_gather(adst_v, [dg]))
                ex[pl.ds(g * _L, _L)] = jnp.exp(_leaky(a) - cval)

            @pl.loop(0, _K // _L)
            def _(gi):
                r0v = gi * _L
                for i in range(_L):
                    w = plsc.load_gather(
                        ex, [jnp.full((_L,), r0v + i, jnp.int32)])
                    for j in range(D // _L):
                        sl = pl.ds(j * _L, _L)
                        rx[r0v + i, sl] = rx[r0v + i, sl] * w

            pltpu.sync_copy(rx, num_sh.at[dx], add=True)
            pltpu.sync_copy(ex, den_sh.at[dx], add=True)
            if not tail:
                issue_idx(sx, dx, six, cq + 2)

        issue_idx(sidx_a, didx_a, sem_ia, 0)
        issue_idx(sidx_b, didx_b, sem_ib, 1)
        wait_idx(sidx_a, didx_a, sem_ia)
        pltpu.async_copy(h_hbm.at[sidx_a], rows_a, sem_ga)

        @pl.loop(0, (nchunks - 1) // 2)
        def _(t):
            cq = t * 2
            process(sidx_a, didx_a, rows_a, ee_a, sem_ia, sem_ga,
                    sidx_b, didx_b, rows_b, sem_ib, sem_gb, cq, False)
            process(sidx_b, didx_b, rows_b, ee_b, sem_ib, sem_gb,
                    sidx_a, didx_a, rows_a, sem_ia, sem_ga, cq + 1, False)

        process(sidx_a, didx_a, rows_a, ee_a, sem_ia, sem_ga,
                sidx_b, didx_b, rows_b, sem_ib, sem_gb, nchunks - 1, True)
        wait_idx(sidx_b, didx_b, sem_ib)

        plsc.subcore_barrier()

        for cc, (num_hbm, den_hbm) in enumerate(
                [(num0_hbm, den0_hbm), (num1_hbm, den1_hbm)]):
            @pl.when(jnp.logical_and(ci == cc, si < 15))
            def _():
                pltpu.sync_copy(num_sh.at[pl.ds(r0, stripe)],
                                num_hbm.at[pl.ds(r0, stripe)])

            @pl.when(jnp.logical_and(ci == cc, si == 15))
            def _():
                pltpu.sync_copy(num_sh.at[pl.ds(r0, last)],
                                num_hbm.at[pl.ds(r0, last)])

            @pl.when(jnp.logical_and(ci == cc, si == 15))
            def _():
                pltpu.sync_copy(den_sh, den_hbm)

    return edge_kernel(src, dst, asrc, adst, cvec, h)


def kernel(x, edge_index, W1, a_src1, a_dst1, b1, W2, a_src2, a_dst2, b2):
    N = x.shape[0]
    R = 2000
    src = edge_index[0]
    dst = edge_index[1]

    h1, s1, t1, mx1 = _proj(x, W1, a_src1.reshape(-1, 1),
                            a_dst1.reshape(-1, 1), R)
    c1 = _leaky(mx1[0, 0] + mx1[0, 1])
    n0, n1, d0, d1 = _edge_aggregate(src, dst, s1.reshape(-1), t1.reshape(-1),
                                     jnp.full((_L,), c1, jnp.float32), h1)
    h2, s2, t2, mx2 = _combine_proj(n0, n1, d0.reshape(N, 1),
                                    d1.reshape(N, 1), s1, t1,
                                    c1.reshape(1, 1), h1, b1.reshape(1, -1),
                                    W2, a_src2.reshape(-1, 1),
                                    a_dst2.reshape(-1, 1), R)
    c2 = _leaky(mx2[0, 0] + mx2[0, 1])
    n0, n1, d0, d1 = _edge_aggregate(src, dst, s2.reshape(-1), t2.reshape(-1),
                                     jnp.full((_L,), c2, jnp.float32), h2)
    return _final(n0, n1, d0.reshape(N, 1), d1.reshape(N, 1), s2, t2,
                  c2.reshape(1, 1), h2, b2.reshape(1, -1), R)


# speedup vs baseline: 46.3147x; 1.2627x over previous
import dataclasses
import functools

import jax
import jax.numpy as jnp
from jax import lax
from jax.experimental import pallas as pl
from jax.experimental.pallas import tpu as pltpu
from jax.experimental.pallas import tpu_sc as plsc

_L = 16
_K = 80
_NC = 2
_NS = 16
_NW = _NC * _NS


def _leaky(a):
    return jnp.maximum(a, 0.2 * a)


def _proj_body(x_ref, w_ref, av_ref, bv_ref, h_ref, s_ref, t_ref, mx_ref):
    h = jnp.dot(x_ref[...], w_ref[...], preferred_element_type=jnp.float32)
    h_ref[...] = h
    sv = jnp.dot(h, av_ref[...], preferred_element_type=jnp.float32)
    tv = jnp.dot(h, bv_ref[...], preferred_element_type=jnp.float32)
    s_ref[...] = sv
    t_ref[...] = tv
    m = jnp.concatenate(
        [jnp.max(sv).reshape(1, 1), jnp.max(tv).reshape(1, 1)], axis=1)

    @pl.when(pl.program_id(0) == 0)
    def _():
        mx_ref[...] = m

    @pl.when(pl.program_id(0) != 0)
    def _():
        mx_ref[...] = jnp.maximum(mx_ref[...], m)


def _proj(x, W, av, bv, R):
    N, Din = x.shape
    D = W.shape[1]
    return pl.pallas_call(
        _proj_body,
        grid=(N // R,),
        in_specs=[
            pl.BlockSpec((R, Din), lambda i: (i, 0)),
            pl.BlockSpec((Din, D), lambda i: (0, 0)),
            pl.BlockSpec((D, 1), lambda i: (0, 0)),
            pl.BlockSpec((D, 1), lambda i: (0, 0)),
        ],
        out_specs=[
            pl.BlockSpec((R, D), lambda i: (i, 0)),
            pl.BlockSpec((R, 1), lambda i: (i, 0)),
            pl.BlockSpec((R, 1), lambda i: (i, 0)),
            pl.BlockSpec((1, 2), lambda i: (0, 0)),
        ],
        out_shape=[
            jax.ShapeDtypeStruct((N, D), jnp.float32),
            jax.ShapeDtypeStruct((N, 1), jnp.float32),
            jax.ShapeDtypeStruct((N, 1), jnp.float32),
            jax.ShapeDtypeStruct((1, 2), jnp.float32),
        ],
    )(x, W, av, bv)


def _merge(n0_ref, n1_ref, d0_ref, d1_ref, s_ref, t_ref, c_ref, h_ref):
    a = s_ref[...] + t_ref[...]
    es = jnp.exp(_leaky(a) - c_ref[0, 0])
    num = n0_ref[...] + n1_ref[...] + es * h_ref[...]
    den = d0_ref[...] + d1_ref[...] + es
    return num / den


def _combine_proj_body(n0_ref, n1_ref, d0_ref, d1_ref, s_ref, t_ref, c_ref,
                       h_ref, b_ref, w_ref, av_ref, bv_ref, h2_ref, s2_ref,
                       t2_ref, mx_ref):
    out1 = _merge(n0_ref, n1_ref, d0_ref, d1_ref, s_ref, t_ref, c_ref, h_ref)
    x2 = jnp.maximum(out1 + b_ref[...], 0.0)
    h2 = jnp.dot(x2, w_ref[...], preferred_element_type=jnp.float32)
    h2_ref[...] = h2
    sv = jnp.dot(h2, av_ref[...], preferred_element_type=jnp.float32)
    tv = jnp.dot(h2, bv_ref[...], preferred_element_type=jnp.float32)
    s2_ref[...] = sv
    t2_ref[...] = tv
    m = jnp.concatenate(
        [jnp.max(sv).reshape(1, 1), jnp.max(tv).reshape(1, 1)], axis=1)

    @pl.when(pl.program_id(0) == 0)
    def _():
        mx_ref[...] = m

    @pl.when(pl.program_id(0) != 0)
    def _():
        mx_ref[...] = jnp.maximum(mx_ref[...], m)


def _combine_proj(n0, n1, d0, d1, s, t, c, h, b, W, av, bv, R):
    N, D = h.shape
    D2 = W.shape[1]
    return pl.pallas_call(
        _combine_proj_body,
        grid=(N // R,),
        in_specs=[
            pl.BlockSpec((R, D), lambda i: (i, 0)),
            pl.BlockSpec((R, D), lambda i: (i, 0)),
            pl.BlockSpec((R, 1), lambda i: (i, 0)),
            pl.BlockSpec((R, 1), lambda i: (i, 0)),
            pl.BlockSpec((R, 1), lambda i: (i, 0)),
            pl.BlockSpec((R, 1), lambda i: (i, 0)),
            pl.BlockSpec((1, 1), lambda i: (0, 0)),
            pl.BlockSpec((R, D), lambda i: (i, 0)),
            pl.BlockSpec((1, D), lambda i: (0, 0)),
            pl.BlockSpec((D, D2), lambda i: (0, 0)),
            pl.BlockSpec((D2, 1), lambda i: (0, 0)),
            pl.BlockSpec((D2, 1), lambda i: (0, 0)),
        ],
        out_specs=[
            pl.BlockSpec((R, D2), lambda i: (i, 0)),
            pl.BlockSpec((R, 1), lambda i: (i, 0)),
            pl.BlockSpec((R, 1), lambda i: (i, 0)),
            pl.BlockSpec((1, 2), lambda i: (0, 0)),
        ],
        out_shape=[
            jax.ShapeDtypeStruct((N, D2), jnp.float32),
            jax.ShapeDtypeStruct((N, 1), jnp.float32),
            jax.ShapeDtypeStruct((N, 1), jnp.float32),
            jax.ShapeDtypeStruct((1, 2), jnp.float32),
        ],
    )(n0, n1, d0, d1, s, t, c, h, b, W, av, bv)


def _final_body(n0_ref, n1_ref, d0_ref, d1_ref, s_ref, t_ref, c_ref, h_ref,
                b_ref, o_ref):
    o_ref[...] = _merge(n0_ref, n1_ref, d0_ref, d1_ref, s_ref, t_ref, c_ref,
                        h_ref) + b_ref[...]


def _final(n0, n1, d0, d1, s, t, c, h, b, R):
    N, D = h.shape
    return pl.pallas_call(
        _final_body,
        grid=(N // R,),
        in_specs=[
            pl.BlockSpec((R, D), lambda i: (i, 0)),
            pl.BlockSpec((R, D), lambda i: (i, 0)),
            pl.BlockSpec((R, 1), lambda i: (i, 0)),
            pl.BlockSpec((R, 1), lambda i: (i, 0)),
            pl.BlockSpec((R, 1), lambda i: (i, 0)),
            pl.BlockSpec((R, 1), lambda i: (i, 0)),
            pl.BlockSpec((1, 1), lambda i: (0, 0)),
            pl.BlockSpec((R, D), lambda i: (i, 0)),
            pl.BlockSpec((1, D), lambda i: (0, 0)),
        ],
        out_specs=pl.BlockSpec((R, D), lambda i: (i, 0)),
        out_shape=jax.ShapeDtypeStruct((N, D), jnp.float32),
    )(n0, n1, d0, d1, s, t, c, h, b)


def _edge_aggregate(src, dst, asrc, adst, cvec, h):
    N, D = h.shape
    E = src.shape[0]
    epw = E // _NW
    nchunks = epw // _K
    stripe = 640
    last = N - 15 * stripe
    dden = N // 10
    dfull = dden // _K
    drem = dden - dfull * _K
    mesh = plsc.VectorSubcoreMesh(core_axis_name="c", subcore_axis_name="s")
    cp = pltpu.CompilerParams()
    if "needs_layout_passes" in pltpu.CompilerParams.__dataclass_fields__:
        cp = dataclasses.replace(cp, needs_layout_passes=False)

    @functools.partial(
        pl.kernel,
        compiler_params=cp,
        out_type=[
            jax.ShapeDtypeStruct((N, D), jnp.float32),
            jax.ShapeDtypeStruct((N, D), jnp.float32),
            jax.ShapeDtypeStruct((N,), jnp.float32),
            jax.ShapeDtypeStruct((N,), jnp.float32),
        ],
        mesh=mesh,
        scratch_types=(
            [pltpu.VMEM((_K,), jnp.int32)] * 8 +
            [pltpu.VMEM((_K, D), jnp.float32)] * 4 +
            [pltpu.VMEM((_K,), jnp.float32)] * 12 +
            [pltpu.VMEM((_L,), jnp.float32)] +
            [pltpu.VMEM_SHARED((N, D), jnp.float32),
             pltpu.VMEM_SHARED((N,), jnp.float32)] +
            [pltpu.SemaphoreType.DMA] * 12
        ),
    )
    def edge_kernel(src_hbm, dst_hbm, asrc_hbm, adst_hbm, cvec_hbm, h_hbm,
                    num0_hbm, num1_hbm, den0_hbm, den1_hbm, *bufs):
        sidx = list(bufs[0:4])
        didx = list(bufs[4:8])
        rows = list(bufs[8:12])
        ee = list(bufs[12:16])
        av = list(bufs[16:20])
        dv = list(bufs[20:24])
        cvec_v = bufs[24]
        num_sh, den_sh = bufs[25], bufs[26]
        sem_i = list(bufs[27:31])
        sem_g = list(bufs[31:35])
        sem_s = list(bufs[35:39])
        ci = lax.axis_index("c")
        si = lax.axis_index("s")
        wid = ci * _NS + si

        @pl.loop(0, _K)
        def _(r):
            for j in range(D // _L):
                z = jnp.zeros((_L,), jnp.float32)
                rows[0][r, pl.ds(j * _L, _L)] = z
                rows[3][r, pl.ds(j * _L, _L)] = z

        @pl.loop(0, _K, step=_L)
        def _(i):
            ee[0][pl.ds(i, _L)] = jnp.zeros((_L,), jnp.float32)
            ee[3][pl.ds(i, _L)] = jnp.zeros((_L,), jnp.float32)
            didx[3][pl.ds(i, _L)] = jnp.zeros((_L,), jnp.int32)

        r0 = si * stripe
        nzero = jnp.where(si < 15, stripe // _K, last // _K)

        @pl.loop(0, nzero)
        def _(q):
            pltpu.sync_copy(rows[0], num_sh.at[pl.ds(r0 + q * _K, _K)])

        @pl.when(si < 10)
        def _():
            d0 = si * dden

            @pl.loop(0, dfull)
            def _(q):
                pltpu.sync_copy(ee[0], den_sh.at[pl.ds(d0 + q * _K, _K)])

            pltpu.sync_copy(ee[0].at[pl.ds(0, drem)],
                            den_sh.at[pl.ds(d0 + dfull * _K, drem)])

        pltpu.sync_copy(cvec_hbm, cvec_v)
        plsc.subcore_barrier()
        cval = cvec_v[...]

        base = wid * epw

        def issue_idx(k, cq):
            off = base + jnp.minimum(cq, nchunks - 1) * _K
            pltpu.async_copy(src_hbm.at[pl.ds(off, _K)], sidx[k], sem_i[k])
            pltpu.async_copy(dst_hbm.at[pl.ds(off, _K)], didx[k], sem_i[k])

        def wait_idx(k):
            pltpu.make_async_copy(src_hbm.at[pl.ds(0, _K)], sidx[k],
                                  sem_i[k]).wait()
            pltpu.make_async_copy(dst_hbm.at[pl.ds(0, _K)], didx[k],
                                  sem_i[k]).wait()

        def issue_scatter(k):
            pltpu.async_copy(rows[k], num_sh.at[didx[k]], sem_s[k], add=True)
            pltpu.async_copy(ee[k], den_sh.at[didx[k]], sem_s[k], add=True)

        def wait_scatter(k):
            pltpu.make_async_copy(rows[k], num_sh.at[didx[k]],
                                  sem_s[k]).wait()
            pltpu.make_async_copy(ee[k], den_sh.at[didx[k]],
                                  sem_s[k]).wait()

        def issue_gathers(k):
            pltpu.async_copy(h_hbm.at[sidx[k]], rows[k], sem_g[k])
            pltpu.async_copy(asrc_hbm.at[sidx[k]], av[k], sem_g[k])
            pltpu.async_copy(adst_hbm.at[didx[k]], dv[k], sem_g[k])

        def wait_gathers(k):
            pltpu.make_async_copy(h_hbm.at[sidx[k]], rows[k],
                                  sem_g[k]).wait()
            pltpu.make_async_copy(asrc_hbm.at[sidx[k]], av[k],
                                  sem_g[k]).wait()
            pltpu.make_async_copy(adst_hbm.at[didx[k]], dv[k],
                                  sem_g[k]).wait()

        def process(x, cq, tail):
            xn, xp = (x + 1) % 4, (x + 3) % 4
            wait_gathers(x)

            if not tail:
                wait_idx(xn)
                issue_gathers(xn)

            for g in range(_K // _L):
                sl = pl.ds(g * _L, _L)
                a = av[x][sl] + dv[x][sl]
                ee[x][sl] = jnp.exp(_leaky(a) - cval)

            @pl.loop(0, _K // _L)
            def _(gi):
                r0v = gi * _L
                for i in range(_L):
                    w = plsc.load_gather(
                        ee[x], [jnp.full((_L,), r0v + i, jnp.int32)])
                    for j in range(D // _L):
                        sl = pl.ds(j * _L, _L)
                        rows[x][r0v + i, sl] = rows[x][r0v + i, sl] * w

            issue_scatter(x)
            wait_scatter(xp)
            if not tail:
                issue_idx(xp, cq + 3)

        issue_scatter(3)
        issue_idx(0, 0)
        issue_idx(1, 1)
        issue_idx(2, 2)
        wait_idx(0)
        issue_gathers(0)

        @pl.loop(0, nchunks // 4)
        def _(t):
            cq = t * 4
            for x in range(4):
                process(x, cq + x, False)

        process(0, nchunks - 1, True)
        wait_scatter(0)
        wait_idx(1)
        wait_idx(2)

        plsc.subcore_barrier()

        for cc, (num_hbm, den_hbm) in enumerate(
                [(num0_hbm, den0_hbm), (num1_hbm, den1_hbm)]):
            @pl.when(jnp.logical_and(ci == cc, si < 15))
            def _():
                pltpu.sync_copy(num_sh.at[pl.ds(r0, stripe)],
                                num_hbm.at[pl.ds(r0, stripe)])

            @pl.when(jnp.logical_and(ci == cc, si == 15))
            def _():
                pltpu.sync_copy(num_sh.at[pl.ds(r0, last)],
                                num_hbm.at[pl.ds(r0, last)])

            @pl.when(jnp.logical_and(ci == cc, si == 15))
            def _():
                pltpu.sync_copy(den_sh, den_hbm)

    return edge_kernel(src, dst, asrc, adst, cvec, h)


def kernel(x, edge_index, W1, a_src1, a_dst1, b1, W2, a_src2, a_dst2, b2):
    N = x.shape[0]
    R = 2000
    src = edge_index[0]
    dst = edge_index[1]

    h1, s1, t1, mx1 = _proj(x, W1, a_src1.reshape(-1, 1),
                            a_dst1.reshape(-1, 1), R)
    c1 = _leaky(mx1[0, 0] + mx1[0, 1])
    n0, n1, d0, d1 = _edge_aggregate(src, dst, s1.reshape(-1), t1.reshape(-1),
                                     jnp.full((_L,), c1, jnp.float32), h1)
    h2, s2, t2, mx2 = _combine_proj(n0, n1, d0.reshape(N, 1),
                                    d1.reshape(N, 1), s1, t1,
                                    c1.reshape(1, 1), h1, b1.reshape(1, -1),
                                    W2, a_src2.reshape(-1, 1),
                                    a_dst2.reshape(-1, 1), R)
    c2 = _leaky(mx2[0, 0] + mx2[0, 1])
    n0, n1, d0, d1 = _edge_aggregate(src, dst, s2.reshape(-1), t2.reshape(-1),
                                     jnp.full((_L,), c2, jnp.float32), h2)
    return _final(n0, n1, d0.reshape(N, 1), d1.reshape(N, 1), s2, t2,
                  c2.reshape(1, 1), h2, b2.reshape(1, -1), R)


# speedup vs baseline: 48.0465x; 1.0374x over previous
import dataclasses
import functools

import jax
import jax.numpy as jnp
from jax import lax
from jax.experimental import pallas as pl
from jax.experimental.pallas import tpu as pltpu
from jax.experimental.pallas import tpu_sc as plsc

_L = 16
_K = 80
_NC = 2
_NS = 16
_NW = _NC * _NS


def _leaky(a):
    return jnp.maximum(a, 0.2 * a)


def _mx_update(mx_ref, cv_ref, sv, tv):
    m = jnp.concatenate(
        [jnp.max(sv).reshape(1, 1), jnp.max(tv).reshape(1, 1)], axis=1)

    @pl.when(pl.program_id(0) == 0)
    def _():
        mx_ref[...] = m

    @pl.when(pl.program_id(0) != 0)
    def _():
        mx_ref[...] = jnp.maximum(mx_ref[...], m)

    @pl.when(pl.program_id(0) == pl.num_programs(0) - 1)
    def _():
        c = _leaky(mx_ref[0, 0] + mx_ref[0, 1])
        cv_ref[...] = jnp.full((1, _L), c, jnp.float32)


def _proj_body(x_ref, w_ref, av_ref, bv_ref, h_ref, s_ref, t_ref, mx_ref,
               cv_ref):
    h = jnp.dot(x_ref[...], w_ref[...], preferred_element_type=jnp.float32)
    h_ref[...] = h
    sv = jnp.dot(h, av_ref[...], preferred_element_type=jnp.float32)
    tv = jnp.dot(h, bv_ref[...], preferred_element_type=jnp.float32)
    s_ref[...] = sv
    t_ref[...] = tv
    _mx_update(mx_ref, cv_ref, sv, tv)


def _proj(x, W, av, bv, R):
    N, Din = x.shape
    D = W.shape[1]
    return pl.pallas_call(
        _proj_body,
        grid=(N // R,),
        in_specs=[
            pl.BlockSpec((R, Din), lambda i: (i, 0)),
            pl.BlockSpec((Din, D), lambda i: (0, 0)),
            pl.BlockSpec((D, 1), lambda i: (0, 0)),
            pl.BlockSpec((D, 1), lambda i: (0, 0)),
        ],
        out_specs=[
            pl.BlockSpec((R, D), lambda i: (i, 0)),
            pl.BlockSpec((R, 1), lambda i: (i, 0)),
            pl.BlockSpec((R, 1), lambda i: (i, 0)),
            pl.BlockSpec((1, 2), lambda i: (0, 0)),
            pl.BlockSpec((1, _L), lambda i: (0, 0)),
        ],
        out_shape=[
            jax.ShapeDtypeStruct((N, D), jnp.float32),
            jax.ShapeDtypeStruct((N, 1), jnp.float32),
            jax.ShapeDtypeStruct((N, 1), jnp.float32),
            jax.ShapeDtypeStruct((1, 2), jnp.float32),
            jax.ShapeDtypeStruct((1, _L), jnp.float32),
        ],
    )(x, W, av, bv)


def _merge(n0_ref, n1_ref, d0_ref, d1_ref, s_ref, t_ref, c_ref, h_ref):
    a = s_ref[...] + t_ref[...]
    es = jnp.exp(_leaky(a) - c_ref[0, 0])
    num = n0_ref[...] + n1_ref[...] + es * h_ref[...]
    den = d0_ref[...] + d1_ref[...] + es
    return num / den


def _combine_proj_body(n0_ref, n1_ref, d0_ref, d1_ref, s_ref, t_ref, c_ref,
                       h_ref, b_ref, w_ref, av_ref, bv_ref, h2_ref, s2_ref,
                       t2_ref, mx_ref, cv_ref):
    out1 = _merge(n0_ref, n1_ref, d0_ref, d1_ref, s_ref, t_ref, c_ref, h_ref)
    x2 = jnp.maximum(out1 + b_ref[...], 0.0)
    h2 = jnp.dot(x2, w_ref[...], preferred_element_type=jnp.float32)
    h2_ref[...] = h2
    sv = jnp.dot(h2, av_ref[...], preferred_element_type=jnp.float32)
    tv = jnp.dot(h2, bv_ref[...], preferred_element_type=jnp.float32)
    s2_ref[...] = sv
    t2_ref[...] = tv
    _mx_update(mx_ref, cv_ref, sv, tv)


def _combine_proj(n0, n1, d0, d1, s, t, c, h, b, W, av, bv, R):
    N, D = h.shape
    D2 = W.shape[1]
    return pl.pallas_call(
        _combine_proj_body,
        grid=(N // R,),
        in_specs=[
            pl.BlockSpec((R, D), lambda i: (i, 0)),
            pl.BlockSpec((R, D), lambda i: (i, 0)),
            pl.BlockSpec((R, 1), lambda i: (i, 0)),
            pl.BlockSpec((R, 1), lambda i: (i, 0)),
            pl.BlockSpec((R, 1), lambda i: (i, 0)),
            pl.BlockSpec((R, 1), lambda i: (i, 0)),
            pl.BlockSpec((1, _L), lambda i: (0, 0)),
            pl.BlockSpec((R, D), lambda i: (i, 0)),
            pl.BlockSpec((1, D), lambda i: (0, 0)),
            pl.BlockSpec((D, D2), lambda i: (0, 0)),
            pl.BlockSpec((D2, 1), lambda i: (0, 0)),
            pl.BlockSpec((D2, 1), lambda i: (0, 0)),
        ],
        out_specs=[
            pl.BlockSpec((R, D2), lambda i: (i, 0)),
            pl.BlockSpec((R, 1), lambda i: (i, 0)),
            pl.BlockSpec((R, 1), lambda i: (i, 0)),
            pl.BlockSpec((1, 2), lambda i: (0, 0)),
            pl.BlockSpec((1, _L), lambda i: (0, 0)),
        ],
        out_shape=[
            jax.ShapeDtypeStruct((N, D2), jnp.float32),
            jax.ShapeDtypeStruct((N, 1), jnp.float32),
            jax.ShapeDtypeStruct((N, 1), jnp.float32),
            jax.ShapeDtypeStruct((1, 2), jnp.float32),
            jax.ShapeDtypeStruct((1, _L), jnp.float32),
        ],
    )(n0, n1, d0, d1, s, t, c, h, b, W, av, bv)


def _final_body(n0_ref, n1_ref, d0_ref, d1_ref, s_ref, t_ref, c_ref, h_ref,
                b_ref, o_ref):
    o_ref[...] = _merge(n0_ref, n1_ref, d0_ref, d1_ref, s_ref, t_ref, c_ref,
                        h_ref) + b_ref[...]


def _final(n0, n1, d0, d1, s, t, c, h, b, R):
    N, D = h.shape
    return pl.pallas_call(
        _final_body,
        grid=(N // R,),
        in_specs=[
            pl.BlockSpec((R, D), lambda i: (i, 0)),
            pl.BlockSpec((R, D), lambda i: (i, 0)),
            pl.BlockSpec((R, 1), lambda i: (i, 0)),
            pl.BlockSpec((R, 1), lambda i: (i, 0)),
            pl.BlockSpec((R, 1), lambda i: (i, 0)),
            pl.BlockSpec((R, 1), lambda i: (i, 0)),
            pl.BlockSpec((1, _L), lambda i: (0, 0)),
            pl.BlockSpec((R, D), lambda i: (i, 0)),
            pl.BlockSpec((1, D), lambda i: (0, 0)),
        ],
        out_specs=pl.BlockSpec((R, D), lambda i: (i, 0)),
        out_shape=jax.ShapeDtypeStruct((N, D), jnp.float32),
    )(n0, n1, d0, d1, s, t, c, h, b)


def _edge_aggregate(src, dst, asrc, adst, cvec, h):
    N, D = h.shape
    E = src.shape[0]
    epw = E // _NW
    nchunks = epw // _K
    stripe = 640
    last = N - 15 * stripe
    dden = N // 10
    dfull = dden // _K
    drem = dden - dfull * _K
    mesh = plsc.VectorSubcoreMesh(core_axis_name="c", subcore_axis_name="s")
    cp = pltpu.CompilerParams()
    if "needs_layout_passes" in pltpu.CompilerParams.__dataclass_fields__:
        cp = dataclasses.replace(cp, needs_layout_passes=False)

    @functools.partial(
        pl.kernel,
        compiler_params=cp,
        out_type=[
            jax.ShapeDtypeStruct((N, D), jnp.float32),
            jax.ShapeDtypeStruct((N, D), jnp.float32),
            jax.ShapeDtypeStruct((N,), jnp.float32),
            jax.ShapeDtypeStruct((N,), jnp.float32),
        ],
        mesh=mesh,
        scratch_types=(
            [pltpu.VMEM((_K,), jnp.int32)] * 8 +
            [pltpu.VMEM((_K, D), jnp.float32)] * 4 +
            [pltpu.VMEM((_K,), jnp.float32)] * 12 +
            [pltpu.VMEM((_L,), jnp.float32)] +
            [pltpu.VMEM_SHARED((N, D), jnp.float32),
             pltpu.VMEM_SHARED((N,), jnp.float32)] +
            [pltpu.SemaphoreType.DMA] * 12
        ),
    )
    def edge_kernel(src_hbm, dst_hbm, asrc_hbm, adst_hbm, cvec_hbm, h_hbm,
                    num0_hbm, num1_hbm, den0_hbm, den1_hbm, *bufs):
        sidx = list(bufs[0:4])
        didx = list(bufs[4:8])
        rows = list(bufs[8:12])
        ee = list(bufs[12:16])
        av = list(bufs[16:20])
        dv = list(bufs[20:24])
        cvec_v = bufs[24]
        num_sh, den_sh = bufs[25], bufs[26]
        sem_i = list(bufs[27:31])
        sem_g = list(bufs[31:35])
        sem_s = list(bufs[35:39])
        ci = lax.axis_index("c")
        si = lax.axis_index("s")
        wid = ci * _NS + si

        @pl.loop(0, _K)
        def _(r):
            for j in range(D // _L):
                z = jnp.zeros((_L,), jnp.float32)
                rows[0][r, pl.ds(j * _L, _L)] = z
                rows[3][r, pl.ds(j * _L, _L)] = z

        @pl.loop(0, _K, step=_L)
        def _(i):
            ee[0][pl.ds(i, _L)] = jnp.zeros((_L,), jnp.float32)
            ee[3][pl.ds(i, _L)] = jnp.zeros((_L,), jnp.float32)
            didx[3][pl.ds(i, _L)] = jnp.zeros((_L,), jnp.int32)

        r0 = si * stripe
        nzero = jnp.where(si < 15, stripe // _K, last // _K)

        @pl.loop(0, nzero)
        def _(q):
            pltpu.sync_copy(rows[0], num_sh.at[pl.ds(r0 + q * _K, _K)])

        @pl.when(si < 10)
        def _():
            d0 = si * dden

            @pl.loop(0, dfull)
            def _(q):
                pltpu.sync_copy(ee[0], den_sh.at[pl.ds(d0 + q * _K, _K)])

            pltpu.sync_copy(ee[0].at[pl.ds(0, drem)],
                            den_sh.at[pl.ds(d0 + dfull * _K, drem)])

        pltpu.sync_copy(cvec_hbm, cvec_v)
        plsc.subcore_barrier()
        cval = cvec_v[...]

        base = wid * epw

        def issue_idx(k, cq):
            off = base + jnp.minimum(cq, nchunks - 1) * _K
            pltpu.async_copy(src_hbm.at[pl.ds(off, _K)], sidx[k], sem_i[k])
            pltpu.async_copy(dst_hbm.at[pl.ds(off, _K)], didx[k], sem_i[k])

        def wait_idx(k):
            pltpu.make_async_copy(src_hbm.at[pl.ds(0, _K)], sidx[k],
                                  sem_i[k]).wait()
            pltpu.make_async_copy(dst_hbm.at[pl.ds(0, _K)], didx[k],
                                  sem_i[k]).wait()

        def issue_scatter(k):
            pltpu.async_copy(rows[k], num_sh.at[didx[k]], sem_s[k], add=True)
            pltpu.async_copy(ee[k], den_sh.at[didx[k]], sem_s[k], add=True)

        def wait_scatter(k):
            pltpu.make_async_copy(rows[k], num_sh.at[didx[k]],
                                  sem_s[k]).wait()
            pltpu.make_async_copy(ee[k], den_sh.at[didx[k]],
                                  sem_s[k]).wait()

        def issue_gathers(k):
            pltpu.async_copy(h_hbm.at[sidx[k]], rows[k], sem_g[k])
            pltpu.async_copy(asrc_hbm.at[sidx[k]], av[k], sem_g[k])
            pltpu.async_copy(adst_hbm.at[didx[k]], dv[k], sem_g[k])

        def wait_gathers(k):
            pltpu.make_async_copy(h_hbm.at[sidx[k]], rows[k],
                                  sem_g[k]).wait()
            pltpu.make_async_copy(asrc_hbm.at[sidx[k]], av[k],
                                  sem_g[k]).wait()
            pltpu.make_async_copy(adst_hbm.at[didx[k]], dv[k],
                                  sem_g[k]).wait()

        def process(x, cq, tail):
            xn, xp = (x + 2) % 4, (x + 3) % 4
            wait_gathers(x)

            if not tail:
                wait_idx(xn)
                issue_gathers(xn)

            for g in range(_K // _L):
                sl = pl.ds(g * _L, _L)
                a = av[x][sl] + dv[x][sl]
                ee[x][sl] = jnp.exp(_leaky(a) - cval)

            @pl.loop(0, _K // _L)
            def _(gi):
                r0v = gi * _L
                for i in range(_L):
                    w = plsc.load_gather(
                        ee[x], [jnp.full((_L,), r0v + i, jnp.int32)])
                    for j in range(D // _L):
                        sl = pl.ds(j * _L, _L)
                        rows[x][r0v + i, sl] = rows[x][r0v + i, sl] * w

            issue_scatter(x)
            wait_scatter(xp)
            if not tail:
                issue_idx(xp, cq + 3)

        issue_scatter(3)
        issue_idx(0, 0)
        issue_idx(1, 1)
        issue_idx(2, 2)
        wait_idx(0)
        issue_gathers(0)
        wait_idx(1)
        issue_gathers(1)

        @pl.loop(0, nchunks // 4)
        def _(t):
            cq = t * 4
            for x in range(4):
                process(x, cq + x, False)

        process(0, nchunks - 1, True)
        wait_scatter(0)
        wait_gathers(1)
        wait_idx(2)

        plsc.subcore_barrier()

        for cc, (num_hbm, den_hbm) in enumerate(
                [(num0_hbm, den0_hbm), (num1_hbm, den1_hbm)]):
            @pl.when(jnp.logical_and(ci == cc, si < 15))
            def _():
                pltpu.sync_copy(num_sh.at[pl.ds(r0, stripe)],
                                num_hbm.at[pl.ds(r0, stripe)])

            @pl.when(jnp.logical_and(ci == cc, si == 15))
            def _():
                pltpu.sync_copy(num_sh.at[pl.ds(r0, last)],
                                num_hbm.at[pl.ds(r0, last)])

            @pl.when(jnp.logical_and(ci == cc, si == 15))
            def _():
                pltpu.sync_copy(den_sh, den_hbm)

    return edge_kernel(src, dst, asrc, adst, cvec, h)


def kernel(x, edge_index, W1, a_src1, a_dst1, b1, W2, a_src2, a_dst2, b2):
    N = x.shape[0]
    R = 2000
    src = edge_index[0]
    dst = edge_index[1]

    h1, s1, t1, mx1, cv1 = _proj(x, W1, a_src1.reshape(-1, 1),
                                 a_dst1.reshape(-1, 1), R)
    n0, n1, d0, d1 = _edge_aggregate(src, dst, s1.reshape(-1), t1.reshape(-1),
                                     cv1.reshape(-1), h1)
    h2, s2, t2, mx2, cv2 = _combine_proj(n0, n1, d0.reshape(N, 1),
                                         d1.reshape(N, 1), s1, t1, cv1, h1,
                                         b1.reshape(1, -1), W2,
                                         a_src2.reshape(-1, 1),
                                         a_dst2.reshape(-1, 1), R)
    n0, n1, d0, d1 = _edge_aggregate(src, dst, s2.reshape(-1), t2.reshape(-1),
                                     cv2.reshape(-1), h2)
    return _final(n0, n1, d0.reshape(N, 1), d1.reshape(N, 1), s2, t2, cv2,
                  h2, b2.reshape(1, -1), R)
